# baseline (speedup 1.0000x reference)
import functools

import jax
import jax.numpy as jnp
import numpy as np
from jax import lax
from jax.experimental import pallas as pl
from jax.experimental.pallas import tpu as pltpu
from jax.experimental.pallas import tpu_sc as plsc

N = 10000
E = 160000
D = 256
HALF = 128
EPAD = 163840
EROWS = EPAD // 128
NP = 10240
NB_ROW = 25
RB = 400

_MESH = dict(core_axis_name="c", subcore_axis_name="s")


def _hist_body(row2d, out, idxb, valb, zb, sem, shared):
    c = lax.axis_index("c")
    s = lax.axis_index("s")
    wid = c * 16 + s

    def zfill(i, _):
        zb[pl.ds(i * 16, 16)] = jnp.zeros((16,), jnp.float32)
        return 0

    lax.fori_loop(0, 40, zfill, 0)
    pltpu.sync_copy(zb, shared.at[pl.ds(s * 640, 640)])
    pltpu.sync_copy(row2d.at[pl.ds(wid * 40, 40)], idxb)
    plsc.subcore_barrier()

    def batch(g, _):
        base = (wid * 40 + g) * 128
        for k in range(8):
            pos = base + k * 16 + lax.iota(jnp.int32, 16)
            valb[pl.ds(k * 16, 16)] = jnp.where(pos < E, 1.0, 0.0)
        pltpu.sync_copy(valb, shared.at[idxb.at[g]], add=True)
        return 0

    lax.fori_loop(0, 40, batch, 0)
    plsc.subcore_barrier()
    pltpu.sync_copy(shared.at[pl.ds(s * 640, 640)],
                    out.at[c, pl.ds(s * 640, 640)])


def _degree_hist(row2d_un):
    return pl.kernel(
        _hist_body,
        out_type=jax.ShapeDtypeStruct((2, NP), jnp.float32),
        mesh=plsc.VectorSubcoreMesh(**_MESH),
        scratch_types=[
            pltpu.VMEM((40, 128), jnp.int32),
            pltpu.VMEM((128,), jnp.float32),
            pltpu.VMEM((640,), jnp.float32),
            pltpu.SemaphoreType.DMA,
            pltpu.VMEM_SHARED((NP,), jnp.float32),
        ],
    )(row2d_un)


def _dis_body(p_ref, dis_ref, inv_ref):
    p = p_ref[...]
    deg = p[:, 0:1] + p[:, 1:2] + 1.0
    dis_ref[...] = lax.rsqrt(deg)
    inv_ref[...] = 1.0 / deg


def _dis_invdeg(partial_t):
    return pl.pallas_call(
        _dis_body,
        grid=(8,),
        in_specs=[pl.BlockSpec((NP // 8, 2), lambda j: (j, 0))],
        out_specs=[pl.BlockSpec((NP // 8, 1), lambda j: (j, 0)),
                   pl.BlockSpec((NP // 8, 1), lambda j: (j, 0))],
        out_shape=[jax.ShapeDtypeStruct((NP, 1), jnp.float32),
                   jax.ShapeDtypeStruct((NP, 1), jnp.float32)],
    )(partial_t)


def _norm_body(dis_hbm, row2d, col2d, out, disv, rb, cb, nb, sem):
    c = lax.axis_index("c")
    s = lax.axis_index("s")
    wid = c * 16 + s
    pltpu.sync_copy(dis_hbm, disv)
    pltpu.sync_copy(row2d.at[pl.ds(wid * 40, 40)], rb)
    pltpu.sync_copy(col2d.at[pl.ds(wid * 40, 40)], cb)

    def batch(g, _):
        for k in range(8):
            ri = rb[g, pl.ds(k * 16, 16)]
            ci = cb[g, pl.ds(k * 16, 16)]
            a = plsc.load_gather(disv, [ri >> 7, ri & 127])
            bb = plsc.load_gather(disv, [ci >> 7, ci & 127])
            pos = (wid * 40 + g) * 128 + k * 16 + lax.iota(jnp.int32, 16)
            nb[g, pl.ds(k * 16, 16)] = jnp.where(pos < E, a * bb, 0.0)
        return 0

    lax.fori_loop(0, 40, batch, 0)
    pltpu.sync_copy(nb, out.at[pl.ds(wid * 40, 40)])


def _edge_norm(dis_flat, row2d, col2d):
    return pl.kernel(
        _norm_body,
        out_type=jax.ShapeDtypeStruct((EROWS, 128), jnp.float32),
        compiler_params=pltpu.CompilerParams(needs_layout_passes=False),
        mesh=plsc.VectorSubcoreMesh(**_MESH),
        scratch_types=[
            pltpu.VMEM((NP // 128, 128), jnp.float32),
            pltpu.VMEM((40, 128), jnp.int32),
            pltpu.VMEM((40, 128), jnp.int32),
            pltpu.VMEM((40, 128), jnp.float32),
            pltpu.SemaphoreType.DMA,
        ],
    )(dis_flat, row2d, col2d)


def _edge_body(xlflat, meta, e8f, agg,
               rows0, rows1, mrows0, mrows1, metav, e8v,
               semg0, semg1, sems0, sems1, shared):
    c = lax.axis_index("c")
    s = lax.axis_index("s")

    pltpu.sync_copy(e8f.at[pl.ds(c * 8, 8)], e8v)

    def zfill(r, _):
        for k in range(8):
            mrows0[r, pl.ds(k * 16, 16)] = jnp.zeros((16,), jnp.float32)
        return 0

    lax.fori_loop(0, 64, zfill, 0)

    @pl.when(s < 15)
    def _():
        for i in range(10):
            pltpu.sync_copy(mrows0, shared.at[pl.ds(s * 640 + i * 64, 64)])

    @pl.when(s == 15)
    def _():
        for i in range(6):
            pltpu.sync_copy(mrows0, shared.at[pl.ds(9600 + i * 64, 64)])
        pltpu.sync_copy(mrows0.at[pl.ds(0, 16)],
                        shared.at[pl.ds(9984, 16)])

    plsc.subcore_barrier()

    lanes = [lax.iota(jnp.int32, 16) + k * 16 for k in range(8)]

    def phase_body(phase, _):
        base = s * 160 + phase * 16
        pltpu.sync_copy(meta.at[pl.ds(base * 5, 80)], metav)
        pltpu.async_copy(xlflat.at[metav.at[c]], rows0, semg0)
        pltpu.async_copy(xlflat.at[metav.at[5 + c]], rows1, semg1)

        def pair(i, _):
            for b, (rows, semg, mrows, sems) in enumerate(
                    ((rows0, semg0, mrows0, sems0),
                     (rows1, semg1, mrows1, sems1))):
                g = 2 * i + b
                pltpu.make_async_copy(xlflat.at[metav.at[5 * g + c]], rows,
                                      semg).wait()

                @pl.when(g >= 2)
                def _():
                    pltpu.make_async_copy(
                        mrows, shared.at[metav.at[2]], sems).wait()

                def grp(j, _):
                    nv16 = plsc.bitcast(metav[5 * g + 3, pl.ds(j * 16, 16)],
                                        jnp.float32)
                    ei16 = metav[5 * g + 4, pl.ds(j * 16, 16)]
                    for t in range(0, 16, 2):
                        e = j * 16 + t
                        tv0 = jnp.full((16,), t, jnp.int32)
                        tv1 = jnp.full((16,), t + 1, jnp.int32)
                        nv0 = nv16.at[tv0].get(mode="promise_in_bounds")
                        ei0 = ei16.at[tv0].get(mode="promise_in_bounds")
                        nv1 = nv16.at[tv1].get(mode="promise_in_bounds")
                        ei1 = ei16.at[tv1].get(mode="promise_in_bounds")
                        ev0 = [plsc.load_gather(e8v, [ei0, lanes[k]])
                               for k in range(8)]
                        ev1 = [plsc.load_gather(e8v, [ei1, lanes[k]])
                               for k in range(8)]
                        xv0 = [rows[e, pl.ds(k * 16, 16)] for k in range(8)]
                        xv1 = [rows[e + 1, pl.ds(k * 16, 16)]
                               for k in range(8)]
                        for k in range(8):
                            mrows[e, pl.ds(k * 16, 16)] = (
                                jnp.maximum(xv0[k] + ev0[k], 0.0) * nv0)
                        for k in range(8):
                            mrows[e + 1, pl.ds(k * 16, 16)] = (
                                jnp.maximum(xv1[k] + ev1[k], 0.0) * nv1)
                    return 0

                lax.fori_loop(0, 4, grp, 0)

                @pl.when(g < 14)
                def _():
                    pltpu.async_copy(xlflat.at[metav.at[5 * (g + 2) + c]],
                                     rows, semg)

                pltpu.async_copy(mrows, shared.at[metav.at[5 * g + 2]],
                                 sems, add=True)
            return 0

        lax.fori_loop(0, 8, pair, 0)
        pltpu.make_async_copy(mrows0, shared.at[metav.at[2]], sems0).wait()
        pltpu.make_async_copy(mrows1, shared.at[metav.at[2]], sems1).wait()
        return 0

    lax.fori_loop(0, 10, phase_body, 0)
    plsc.subcore_barrier()

    @pl.when(s < 15)
    def _():
        pltpu.sync_copy(shared.at[pl.ds(s * 640, 640)],
                        agg.at[c, pl.ds(s * 640, 640)])

    @pl.when(s == 15)
    def _():
        pltpu.sync_copy(shared.at[pl.ds(9600, 400)],
                        agg.at[c, pl.ds(9600, 400)])


def _edge_stage(xlflat, meta, e8f):
    return pl.kernel(
        _edge_body,
        out_type=jax.ShapeDtypeStruct((2, N, HALF), jnp.float32),
        compiler_params=pltpu.CompilerParams(needs_layout_passes=False),
        mesh=plsc.VectorSubcoreMesh(**_MESH),
        scratch_types=[
            pltpu.VMEM((64, 128), jnp.float32),
            pltpu.VMEM((64, 128), jnp.float32),
            pltpu.VMEM((64, 128), jnp.float32),
            pltpu.VMEM((64, 128), jnp.float32),
            pltpu.VMEM((80, 64), jnp.int32),
            pltpu.VMEM((8, 128), jnp.float32),
            pltpu.SemaphoreType.DMA,
            pltpu.SemaphoreType.DMA,
            pltpu.SemaphoreType.DMA,
            pltpu.SemaphoreType.DMA,
            pltpu.VMEM_SHARED((N, HALF), jnp.float32),
        ],
    )(xlflat, meta, e8f)


def _mm0_body(xf_ref, a_ref, c0_ref, w_ref, b_ref, out_ref):
    xf = xf_ref[...]
    h = jnp.broadcast_to(c0_ref[...], (RB, D))
    for i in range(9):
        h = h + xf[:, i:i + 1] * a_ref[i:i + 1, :]
    t = jnp.dot(h, w_ref[...], preferred_element_type=jnp.float32)
    t = t + b_ref[...]
    out_ref[0] = t[:, :HALF]
    out_ref[1] = t[:, HALF:]


def _mm0(xf, a, c0, w, bb):
    return pl.pallas_call(
        _mm0_body,
        grid=(NB_ROW,),
        in_specs=[
            pl.BlockSpec((RB, 9), lambda j: (j, 0)),
            pl.BlockSpec((9, D), lambda j: (0, 0)),
            pl.BlockSpec((1, D), lambda j: (0, 0)),
            pl.BlockSpec((D, D), lambda j: (0, 0)),
            pl.BlockSpec((1, D), lambda j: (0, 0)),
        ],
        out_specs=pl.BlockSpec((2, RB, HALF), lambda j: (0, j, 0)),
        out_shape=jax.ShapeDtypeStruct((2, N, HALF), jnp.float32),
    )(xf, a, c0, w, bb)


def _mm_body(h_ref, sums_ref, g_ref, be_ref, w_ref, b_ref, out_ref):
    sm = sums_ref[...]
    mu = sm[0:1, :] / N
    var = sm[1:2, :] / N - mu * mu
    inv = lax.rsqrt(var + 1e-5)
    h = (h_ref[...] - mu) * inv * g_ref[...] + be_ref[...]
    h = jnp.maximum(h, 0.0)
    t = jnp.dot(h.astype(jnp.bfloat16), w_ref[...].astype(jnp.bfloat16),
                preferred_element_type=jnp.float32)
    t = t + b_ref[...]
    out_ref[0] = t[:, :HALF]
    out_ref[1] = t[:, HALF:]


def _mm(hprev, sums, g, be, w, bb):
    return pl.pallas_call(
        _mm_body,
        grid=(NB_ROW,),
        in_specs=[
            pl.BlockSpec((RB, D), lambda j: (j, 0)),
            pl.BlockSpec((8, D), lambda j: (0, 0)),
            pl.BlockSpec((1, D), lambda j: (0, 0)),
            pl.BlockSpec((1, D), lambda j: (0, 0)),
            pl.BlockSpec((D, D), lambda j: (0, 0)),
            pl.BlockSpec((1, D), lambda j: (0, 0)),
        ],
        out_specs=pl.BlockSpec((2, RB, HALF), lambda j: (0, j, 0)),
        out_shape=jax.ShapeDtypeStruct((2, N, HALF), jnp.float32),
    )(hprev, sums, g, be, w, bb)


def _post_body(agg_ref, xl_ref, root_ref, inv_ref, out_ref, sums_ref):
    a = jnp.concatenate([agg_ref[0], agg_ref[1]], axis=1)
    xl = jnp.concatenate([xl_ref[0], xl_ref[1]], axis=1)
    t = a + jnp.maximum(xl + root_ref[...], 0.0) * inv_ref[...]
    out_ref[...] = t

    @pl.when(pl.program_id(0) == 0)
    def _():
        sums_ref[...] = jnp.zeros_like(sums_ref)

    sums_ref[0:1, :] += jnp.sum(t, axis=0, keepdims=True)
    sums_ref[1:2, :] += jnp.sum(t * t, axis=0, keepdims=True)


def _post(agg, xl, rootl, invdeg):
    return pl.pallas_call(
        _post_body,
        grid=(NB_ROW,),
        in_specs=[
            pl.BlockSpec((2, RB, HALF), lambda j: (0, j, 0)),
            pl.BlockSpec((2, RB, HALF), lambda j: (0, j, 0)),
            pl.BlockSpec((1, D), lambda j: (0, 0)),
            pl.BlockSpec((RB, 1), lambda j: (j, 0)),
        ],
        out_specs=[pl.BlockSpec((RB, D), lambda j: (j, 0)),
                   pl.BlockSpec((8, D), lambda j: (0, 0))],
        out_shape=[jax.ShapeDtypeStruct((N, D), jnp.float32),
                   jax.ShapeDtypeStruct((8, D), jnp.float32)],
    )(agg, xl, rootl, invdeg)


def _final_body(h_ref, sums_ref, g_ref, be_ref, out_ref):
    sm = sums_ref[...]
    mu = sm[0:1, :] / N
    var = sm[1:2, :] / N - mu * mu
    inv = lax.rsqrt(var + 1e-5)
    out_ref[...] = (h_ref[...] - mu) * inv * g_ref[...] + be_ref[...]


def _final(h, sums, g, be):
    return pl.pallas_call(
        _final_body,
        grid=(NB_ROW,),
        in_specs=[
            pl.BlockSpec((RB, D), lambda j: (j, 0)),
            pl.BlockSpec((8, D), lambda j: (0, 0)),
            pl.BlockSpec((1, D), lambda j: (0, 0)),
            pl.BlockSpec((1, D), lambda j: (0, 0)),
        ],
        out_specs=pl.BlockSpec((RB, D), lambda j: (j, 0)),
        out_shape=jax.ShapeDtypeStruct((N, D), jnp.float32),
    )(h, sums, g, be)


def kernel(x, edge_index, edge_attr, atom_tab, W, b, root, bond_tab, gamma,
           beta):
    nl = W.shape[0]
    row = edge_index[0].astype(jnp.int32)
    col = edge_index[1].astype(jnp.int32)

    pad = EPAD - E
    pad_idx = (jnp.arange(pad, dtype=jnp.int32) % N)
    rowp = jnp.concatenate([row, pad_idx])
    colp = jnp.concatenate([col, pad_idx])
    eid = (edge_attr[:, 0] + 2 * edge_attr[:, 1]
           + 4 * edge_attr[:, 2]).astype(jnp.int32)
    eidp = jnp.concatenate([eid, jnp.zeros((pad,), jnp.int32)])

    row2d = rowp.reshape(EROWS, 128)
    col2d = colp.reshape(EROWS, 128)
    row64 = rowp.reshape(EPAD // 64, 64)
    col64 = colp.reshape(EPAD // 64, 64)
    eid64 = eidp.reshape(EPAD // 64, 64)

    a_tab = (atom_tab[:, 1, :] - atom_tab[:, 0, :])
    c0 = atom_tab[:, 0, :].sum(axis=0).reshape(1, D)
    xf = x.astype(jnp.float32)

    kk = np.arange(8)
    e8 = (bond_tab[:, 0, kk & 1, :] + bond_tab[:, 1, (kk >> 1) & 1, :]
          + bond_tab[:, 2, (kk >> 2) & 1, :])
    e8f = jnp.concatenate([e8[:, :, :HALF], e8[:, :, HALF:]], axis=1)

    partial = _degree_hist(row2d)
    dis_col, invdeg = _dis_invdeg(partial.T)
    norm2d = _edge_norm(dis_col.reshape(NP // 128, 128), row2d,
                        col2d)
    normbits = lax.bitcast_convert_type(
        norm2d.reshape(EPAD // 64, 64), jnp.int32)
    meta = jnp.stack([row64, row64 + N, col64, normbits, eid64],
                     axis=1).reshape(EPAD // 64 * 5, 64)

    h = None
    sums = None
    for l in range(nl):
        wl = W[l]
        bl = b[l].reshape(1, D)
        if l == 0:
            xl = _mm0(xf, a_tab, c0, wl, bl)
        else:
            xl = _mm(h, sums, gamma[l - 1].reshape(1, D),
                     beta[l - 1].reshape(1, D), wl, bl)
        agg = _edge_stage(xl.reshape(2 * N, HALF), meta, e8f[l])
        h, sums = _post(agg, xl, root[l].reshape(1, D), invdeg)

    return _final(h, sums, gamma[nl - 1].reshape(1, D),
                  beta[nl - 1].reshape(1, D))


# speedup vs baseline: 1.0416x; 1.0416x over previous
import functools

import jax
import jax.numpy as jnp
import numpy as np
from jax import lax
from jax.experimental import pallas as pl
from jax.experimental.pallas import tpu as pltpu
from jax.experimental.pallas import tpu_sc as plsc

N = 10000
E = 160000
D = 256
HALF = 128
EPAD = 163840
EROWS = EPAD // 128
NP = 10240
NB_ROW = 25
RB = 400

_MESH = dict(core_axis_name="c", subcore_axis_name="s")


def _hist_body(row2d, out, idxb, valb, zb, sem, shared):
    c = lax.axis_index("c")
    s = lax.axis_index("s")
    wid = c * 16 + s

    def zfill(i, _):
        zb[pl.ds(i * 16, 16)] = jnp.zeros((16,), jnp.float32)
        return 0

    lax.fori_loop(0, 40, zfill, 0)
    pltpu.sync_copy(zb, shared.at[pl.ds(s * 640, 640)])
    pltpu.sync_copy(row2d.at[pl.ds(wid * 40, 40)], idxb)
    plsc.subcore_barrier()

    def batch(g, _):
        base = (wid * 40 + g) * 128
        for k in range(8):
            pos = base + k * 16 + lax.iota(jnp.int32, 16)
            valb[pl.ds(k * 16, 16)] = jnp.where(pos < E, 1.0, 0.0)
        pltpu.sync_copy(valb, shared.at[idxb.at[g]], add=True)
        return 0

    lax.fori_loop(0, 40, batch, 0)
    plsc.subcore_barrier()
    pltpu.sync_copy(shared.at[pl.ds(s * 640, 640)],
                    out.at[c, pl.ds(s * 640, 640)])


def _degree_hist(row2d_un):
    return pl.kernel(
        _hist_body,
        out_type=jax.ShapeDtypeStruct((2, NP), jnp.float32),
        mesh=plsc.VectorSubcoreMesh(**_MESH),
        scratch_types=[
            pltpu.VMEM((40, 128), jnp.int32),
            pltpu.VMEM((128,), jnp.float32),
            pltpu.VMEM((640,), jnp.float32),
            pltpu.SemaphoreType.DMA,
            pltpu.VMEM_SHARED((NP,), jnp.float32),
        ],
    )(row2d_un)


def _dis_body(p_ref, dis_ref, inv_ref):
    p = p_ref[...]
    deg = p[:, 0:1] + p[:, 1:2] + 1.0
    dis_ref[...] = lax.rsqrt(deg)
    inv_ref[...] = 1.0 / deg


def _dis_invdeg(partial_t):
    return pl.pallas_call(
        _dis_body,
        grid=(8,),
        in_specs=[pl.BlockSpec((NP // 8, 2), lambda j: (j, 0))],
        out_specs=[pl.BlockSpec((NP // 8, 1), lambda j: (j, 0)),
                   pl.BlockSpec((NP // 8, 1), lambda j: (j, 0))],
        out_shape=[jax.ShapeDtypeStruct((NP, 1), jnp.float32),
                   jax.ShapeDtypeStruct((NP, 1), jnp.float32)],
    )(partial_t)


def _norm_body(dis_hbm, row2d, col2d, out, disv, rb, cb, nb, sem):
    c = lax.axis_index("c")
    s = lax.axis_index("s")
    wid = c * 16 + s
    pltpu.sync_copy(dis_hbm, disv)
    pltpu.sync_copy(row2d.at[pl.ds(wid * 40, 40)], rb)
    pltpu.sync_copy(col2d.at[pl.ds(wid * 40, 40)], cb)

    def batch(g, _):
        for k in range(8):
            ri = rb[g, pl.ds(k * 16, 16)]
            ci = cb[g, pl.ds(k * 16, 16)]
            a = plsc.load_gather(disv, [ri >> 7, ri & 127])
            bb = plsc.load_gather(disv, [ci >> 7, ci & 127])
            pos = (wid * 40 + g) * 128 + k * 16 + lax.iota(jnp.int32, 16)
            nb[g, pl.ds(k * 16, 16)] = jnp.where(pos < E, a * bb, 0.0)
        return 0

    lax.fori_loop(0, 40, batch, 0)
    pltpu.sync_copy(nb, out.at[pl.ds(wid * 40, 40)])


def _edge_norm(dis_flat, row2d, col2d):
    return pl.kernel(
        _norm_body,
        out_type=jax.ShapeDtypeStruct((EROWS, 128), jnp.float32),
        compiler_params=pltpu.CompilerParams(needs_layout_passes=False),
        mesh=plsc.VectorSubcoreMesh(**_MESH),
        scratch_types=[
            pltpu.VMEM((NP // 128, 128), jnp.float32),
            pltpu.VMEM((40, 128), jnp.int32),
            pltpu.VMEM((40, 128), jnp.int32),
            pltpu.VMEM((40, 128), jnp.float32),
            pltpu.SemaphoreType.DMA,
        ],
    )(dis_flat, row2d, col2d)


def _edge_body(xlflat, meta, e8f, agg,
               rows0, rows1, mrows0, mrows1, metav, e8v,
               semg0, semg1, sems0, sems1, shared):
    c = lax.axis_index("c")
    s = lax.axis_index("s")

    pltpu.sync_copy(e8f.at[pl.ds(c * 8, 8)], e8v)

    def zfill(r, _):
        for k in range(8):
            mrows0[r, pl.ds(k * 16, 16)] = jnp.zeros((16,), jnp.float32)
        return 0

    lax.fori_loop(0, 64, zfill, 0)

    @pl.when(s < 15)
    def _():
        for i in range(10):
            pltpu.sync_copy(mrows0, shared.at[pl.ds(s * 640 + i * 64, 64)])

    @pl.when(s == 15)
    def _():
        for i in range(6):
            pltpu.sync_copy(mrows0, shared.at[pl.ds(9600 + i * 64, 64)])
        pltpu.sync_copy(mrows0.at[pl.ds(0, 16)],
                        shared.at[pl.ds(9984, 16)])

    plsc.subcore_barrier()

    lanes = [lax.iota(jnp.int32, 16) + k * 16 for k in range(8)]

    def phase_body(phase, _):
        base = s * 160 + phase * 16
        pltpu.sync_copy(meta.at[pl.ds(base * 5, 80)], metav)
        pltpu.async_copy(xlflat.at[metav.at[c]], rows0, semg0)
        pltpu.async_copy(xlflat.at[metav.at[5 + c]], rows1, semg1)

        def pair(i, _):
            for b, (rows, semg, mrows, sems) in enumerate(
                    ((rows0, semg0, mrows0, sems0),
                     (rows1, semg1, mrows1, sems1))):
                g = 2 * i + b
                pltpu.make_async_copy(xlflat.at[metav.at[5 * g + c]], rows,
                                      semg).wait()

                @pl.when(g >= 2)
                def _():
                    pltpu.make_async_copy(
                        mrows, shared.at[metav.at[2]], sems).wait()

                def grp(j, _):
                    nv16 = plsc.bitcast(metav[5 * g + 3, pl.ds(j * 16, 16)],
                                        jnp.float32)
                    ei16 = metav[5 * g + 4, pl.ds(j * 16, 16)]
                    for t in range(0, 16, 2):
                        e = j * 16 + t
                        tv0 = jnp.full((16,), t, jnp.int32)
                        tv1 = jnp.full((16,), t + 1, jnp.int32)
                        nv0 = nv16.at[tv0].get(mode="promise_in_bounds")
                        ei0 = ei16.at[tv0].get(mode="promise_in_bounds")
                        nv1 = nv16.at[tv1].get(mode="promise_in_bounds")
                        ei1 = ei16.at[tv1].get(mode="promise_in_bounds")
                        ev0 = [plsc.load_gather(e8v, [ei0, lanes[k]])
                               for k in range(8)]
                        ev1 = [plsc.load_gather(e8v, [ei1, lanes[k]])
                               for k in range(8)]
                        xv0 = [rows[e, pl.ds(k * 16, 16)] for k in range(8)]
                        xv1 = [rows[e + 1, pl.ds(k * 16, 16)]
                               for k in range(8)]
                        for k in range(8):
                            mrows[e, pl.ds(k * 16, 16)] = (
                                jnp.maximum(xv0[k] + ev0[k], 0.0) * nv0)
                        for k in range(8):
                            mrows[e + 1, pl.ds(k * 16, 16)] = (
                                jnp.maximum(xv1[k] + ev1[k], 0.0) * nv1)
                    return 0

                lax.fori_loop(0, 4, grp, 0)

                @pl.when(g < 14)
                def _():
                    pltpu.async_copy(xlflat.at[metav.at[5 * (g + 2) + c]],
                                     rows, semg)

                pltpu.async_copy(mrows, shared.at[metav.at[5 * g + 2]],
                                 sems, add=True)
            return 0

        lax.fori_loop(0, 8, pair, 0)
        pltpu.make_async_copy(mrows0, shared.at[metav.at[2]], sems0).wait()
        pltpu.make_async_copy(mrows1, shared.at[metav.at[2]], sems1).wait()
        return 0

    lax.fori_loop(0, 10, phase_body, 0)
    plsc.subcore_barrier()

    @pl.when(s < 15)
    def _():
        pltpu.sync_copy(shared.at[pl.ds(s * 640, 640)],
                        agg.at[c, pl.ds(s * 640, 640)])

    @pl.when(s == 15)
    def _():
        pltpu.sync_copy(shared.at[pl.ds(9600, 400)],
                        agg.at[c, pl.ds(9600, 400)])


def _edge_stage(xlflat, meta, e8f):
    return pl.kernel(
        _edge_body,
        out_type=jax.ShapeDtypeStruct((2, N, HALF), jnp.float32),
        compiler_params=pltpu.CompilerParams(needs_layout_passes=False),
        mesh=plsc.VectorSubcoreMesh(**_MESH),
        scratch_types=[
            pltpu.VMEM((64, 128), jnp.float32),
            pltpu.VMEM((64, 128), jnp.float32),
            pltpu.VMEM((64, 128), jnp.float32),
            pltpu.VMEM((64, 128), jnp.float32),
            pltpu.VMEM((80, 64), jnp.int32),
            pltpu.VMEM((8, 128), jnp.float32),
            pltpu.SemaphoreType.DMA,
            pltpu.SemaphoreType.DMA,
            pltpu.SemaphoreType.DMA,
            pltpu.SemaphoreType.DMA,
            pltpu.VMEM_SHARED((N, HALF), jnp.float32),
        ],
    )(xlflat, meta, e8f)


def _mm0_body(xf_ref, a_ref, c0_ref, w_ref, b_ref, out_ref):
    xf = xf_ref[...]
    h = jnp.broadcast_to(c0_ref[...], (RB, D))
    for i in range(9):
        h = h + xf[:, i:i + 1] * a_ref[i:i + 1, :]
    t = jnp.dot(h, w_ref[...], preferred_element_type=jnp.float32)
    t = t + b_ref[...]
    out_ref[0] = t[:, :HALF]
    out_ref[1] = t[:, HALF:]


def _mm0(xf, a, c0, w, bb):
    return pl.pallas_call(
        _mm0_body,
        grid=(NB_ROW,),
        in_specs=[
            pl.BlockSpec((RB, 9), lambda j: (j, 0)),
            pl.BlockSpec((9, D), lambda j: (0, 0)),
            pl.BlockSpec((1, D), lambda j: (0, 0)),
            pl.BlockSpec((D, D), lambda j: (0, 0)),
            pl.BlockSpec((1, D), lambda j: (0, 0)),
        ],
        out_specs=pl.BlockSpec((2, RB, HALF), lambda j: (0, j, 0)),
        out_shape=jax.ShapeDtypeStruct((2, N, HALF), jnp.float32),
    )(xf, a, c0, w, bb)


def _mm_body(h_ref, sums_ref, g_ref, be_ref, w_ref, b_ref, out_ref):
    sm = sums_ref[...]
    mu = sm[0:1, :] / N
    var = sm[1:2, :] / N - mu * mu
    inv = lax.rsqrt(var + 1e-5)
    h = (h_ref[...] - mu) * inv * g_ref[...] + be_ref[...]
    h = jnp.maximum(h, 0.0)
    t = jnp.dot(h, w_ref[...], preferred_element_type=jnp.float32)
    t = t + b_ref[...]
    out_ref[0] = t[:, :HALF]
    out_ref[1] = t[:, HALF:]


def _mm(hprev, sums, g, be, w, bb):
    return pl.pallas_call(
        _mm_body,
        grid=(NB_ROW,),
        in_specs=[
            pl.BlockSpec((RB, D), lambda j: (j, 0)),
            pl.BlockSpec((8, D), lambda j: (0, 0)),
            pl.BlockSpec((1, D), lambda j: (0, 0)),
            pl.BlockSpec((1, D), lambda j: (0, 0)),
            pl.BlockSpec((D, D), lambda j: (0, 0)),
            pl.BlockSpec((1, D), lambda j: (0, 0)),
        ],
        out_specs=pl.BlockSpec((2, RB, HALF), lambda j: (0, j, 0)),
        out_shape=jax.ShapeDtypeStruct((2, N, HALF), jnp.float32),
    )(hprev, sums, g, be, w, bb)


def _post_body(agg_ref, xl_ref, root_ref, inv_ref, out_ref, sums_ref):
    a = jnp.concatenate([agg_ref[0], agg_ref[1]], axis=1)
    xl = jnp.concatenate([xl_ref[0], xl_ref[1]], axis=1)
    t = a + jnp.maximum(xl + root_ref[...], 0.0) * inv_ref[...]
    out_ref[...] = t

    @pl.when(pl.program_id(0) == 0)
    def _():
        sums_ref[...] = jnp.zeros_like(sums_ref)

    sums_ref[0:1, :] += jnp.sum(t, axis=0, keepdims=True)
    sums_ref[1:2, :] += jnp.sum(t * t, axis=0, keepdims=True)


def _post(agg, xl, rootl, invdeg):
    return pl.pallas_call(
        _post_body,
        grid=(NB_ROW,),
        in_specs=[
            pl.BlockSpec((2, RB, HALF), lambda j: (0, j, 0)),
            pl.BlockSpec((2, RB, HALF), lambda j: (0, j, 0)),
            pl.BlockSpec((1, D), lambda j: (0, 0)),
            pl.BlockSpec((RB, 1), lambda j: (j, 0)),
        ],
        out_specs=[pl.BlockSpec((RB, D), lambda j: (j, 0)),
                   pl.BlockSpec((8, D), lambda j: (0, 0))],
        out_shape=[jax.ShapeDtypeStruct((N, D), jnp.float32),
                   jax.ShapeDtypeStruct((8, D), jnp.float32)],
    )(agg, xl, rootl, invdeg)


def _final_body(h_ref, sums_ref, g_ref, be_ref, out_ref):
    sm = sums_ref[...]
    mu = sm[0:1, :] / N
    var = sm[1:2, :] / N - mu * mu
    inv = lax.rsqrt(var + 1e-5)
    out_ref[...] = (h_ref[...] - mu) * inv * g_ref[...] + be_ref[...]


def _final(h, sums, g, be):
    return pl.pallas_call(
        _final_body,
        grid=(NB_ROW,),
        in_specs=[
            pl.BlockSpec((RB, D), lambda j: (j, 0)),
            pl.BlockSpec((8, D), lambda j: (0, 0)),
            pl.BlockSpec((1, D), lambda j: (0, 0)),
            pl.BlockSpec((1, D), lambda j: (0, 0)),
        ],
        out_specs=pl.BlockSpec((RB, D), lambda j: (j, 0)),
        out_shape=jax.ShapeDtypeStruct((N, D), jnp.float32),
    )(h, sums, g, be)


def _fused_body(agg_ref, xl_ref, root_ref, inv_ref, g_ref, be_ref, w_ref,
                b_ref, out_ref, outbuf, sums):
    j = pl.program_id(0)

    @pl.when(j == 0)
    def _():
        sums[...] = jnp.zeros_like(sums)

    @pl.when(j < NB_ROW)
    def _():
        a = jnp.concatenate([agg_ref[0], agg_ref[1]], axis=1)
        xl = jnp.concatenate([xl_ref[0], xl_ref[1]], axis=1)
        t = a + jnp.maximum(xl + root_ref[...], 0.0) * inv_ref[...]
        outbuf[pl.ds(j * RB, RB), :] = t
        sums[0:1, :] += jnp.sum(t, axis=0, keepdims=True)
        sums[1:2, :] += jnp.sum(t * t, axis=0, keepdims=True)

    @pl.when(j >= NB_ROW)
    def _():
        jj = j - NB_ROW
        o = outbuf[pl.ds(jj * RB, RB), :]
        mu = sums[0:1, :] / N
        var = sums[1:2, :] / N - mu * mu
        ivs = lax.rsqrt(var + 1e-5)
        h = (o - mu) * ivs * g_ref[...] + be_ref[...]
        h = jnp.maximum(h, 0.0)
        t = jnp.dot(h, w_ref[...], preferred_element_type=jnp.float32)
        t = t + b_ref[...]
        out_ref[0] = t[:, :HALF]
        out_ref[1] = t[:, HALF:]


def _fused(agg, xl, rootl, invdeg, g, be, w, bb):
    return pl.pallas_call(
        _fused_body,
        grid=(2 * NB_ROW,),
        in_specs=[
            pl.BlockSpec((2, RB, HALF),
                         lambda j: (0, jnp.where(j < NB_ROW, j, 0), 0)),
            pl.BlockSpec((2, RB, HALF),
                         lambda j: (0, jnp.where(j < NB_ROW, j, 0), 0)),
            pl.BlockSpec((1, D), lambda j: (0, 0)),
            pl.BlockSpec((RB, 1),
                         lambda j: (jnp.where(j < NB_ROW, j, 0), 0)),
            pl.BlockSpec((1, D), lambda j: (0, 0)),
            pl.BlockSpec((1, D), lambda j: (0, 0)),
            pl.BlockSpec((D, D), lambda j: (0, 0)),
            pl.BlockSpec((1, D), lambda j: (0, 0)),
        ],
        out_specs=pl.BlockSpec(
            (2, RB, HALF),
            lambda j: (0, jnp.where(j < NB_ROW, 0, j - NB_ROW), 0)),
        out_shape=jax.ShapeDtypeStruct((2, N, HALF), jnp.float32),
        scratch_shapes=[pltpu.VMEM((N, D), jnp.float32),
                        pltpu.VMEM((8, D), jnp.float32)],
    )(agg, xl, rootl, invdeg, g, be, w, bb)


def _fused_final_body(agg_ref, xl_ref, root_ref, inv_ref, g_ref, be_ref,
                      out_ref, outbuf, sums):
    j = pl.program_id(0)

    @pl.when(j == 0)
    def _():
        sums[...] = jnp.zeros_like(sums)

    @pl.when(j < NB_ROW)
    def _():
        a = jnp.concatenate([agg_ref[0], agg_ref[1]], axis=1)
        xl = jnp.concatenate([xl_ref[0], xl_ref[1]], axis=1)
        t = a + jnp.maximum(xl + root_ref[...], 0.0) * inv_ref[...]
        outbuf[pl.ds(j * RB, RB), :] = t
        sums[0:1, :] += jnp.sum(t, axis=0, keepdims=True)
        sums[1:2, :] += jnp.sum(t * t, axis=0, keepdims=True)

    @pl.when(j >= NB_ROW)
    def _():
        jj = j - NB_ROW
        o = outbuf[pl.ds(jj * RB, RB), :]
        mu = sums[0:1, :] / N
        var = sums[1:2, :] / N - mu * mu
        ivs = lax.rsqrt(var + 1e-5)
        out_ref[...] = (o - mu) * ivs * g_ref[...] + be_ref[...]


def _fused_final(agg, xl, rootl, invdeg, g, be):
    return pl.pallas_call(
        _fused_final_body,
        grid=(2 * NB_ROW,),
        in_specs=[
            pl.BlockSpec((2, RB, HALF),
                         lambda j: (0, jnp.where(j < NB_ROW, j, 0), 0)),
            pl.BlockSpec((2, RB, HALF),
                         lambda j: (0, jnp.where(j < NB_ROW, j, 0), 0)),
            pl.BlockSpec((1, D), lambda j: (0, 0)),
            pl.BlockSpec((RB, 1),
                         lambda j: (jnp.where(j < NB_ROW, j, 0), 0)),
            pl.BlockSpec((1, D), lambda j: (0, 0)),
            pl.BlockSpec((1, D), lambda j: (0, 0)),
        ],
        out_specs=pl.BlockSpec(
            (RB, D),
            lambda j: (jnp.where(j < NB_ROW, 0, j - NB_ROW), 0)),
        out_shape=jax.ShapeDtypeStruct((N, D), jnp.float32),
        scratch_shapes=[pltpu.VMEM((N, D), jnp.float32),
                        pltpu.VMEM((8, D), jnp.float32)],
    )(agg, xl, rootl, invdeg, g, be)


def kernel(x, edge_index, edge_attr, atom_tab, W, b, root, bond_tab, gamma,
           beta):
    nl = W.shape[0]
    row = edge_index[0].astype(jnp.int32)
    col = edge_index[1].astype(jnp.int32)

    pad = EPAD - E
    pad_idx = (jnp.arange(pad, dtype=jnp.int32) % N)
    rowp = jnp.concatenate([row, pad_idx])
    colp = jnp.concatenate([col, pad_idx])
    eid = (edge_attr[:, 0] + 2 * edge_attr[:, 1]
           + 4 * edge_attr[:, 2]).astype(jnp.int32)
    eidp = jnp.concatenate([eid, jnp.zeros((pad,), jnp.int32)])

    row2d = rowp.reshape(EROWS, 128)
    col2d = colp.reshape(EROWS, 128)
    row64 = rowp.reshape(EPAD // 64, 64)
    col64 = colp.reshape(EPAD // 64, 64)
    eid64 = eidp.reshape(EPAD // 64, 64)

    a_tab = (atom_tab[:, 1, :] - atom_tab[:, 0, :])
    c0 = atom_tab[:, 0, :].sum(axis=0).reshape(1, D)
    xf = x.astype(jnp.float32)

    kk = np.arange(8)
    e8 = (bond_tab[:, 0, kk & 1, :] + bond_tab[:, 1, (kk >> 1) & 1, :]
          + bond_tab[:, 2, (kk >> 2) & 1, :])
    e8f = jnp.concatenate([e8[:, :, :HALF], e8[:, :, HALF:]], axis=1)

    partial = _degree_hist(row2d)
    dis_col, invdeg = _dis_invdeg(partial.T)
    norm2d = _edge_norm(dis_col.reshape(NP // 128, 128), row2d,
                        col2d)
    normbits = lax.bitcast_convert_type(
        norm2d.reshape(EPAD // 64, 64), jnp.int32)
    meta = jnp.stack([row64, row64 + N, col64, normbits, eid64],
                     axis=1).reshape(EPAD // 64 * 5, 64)

    xl = _mm0(xf, a_tab, c0, W[0], b[0].reshape(1, D))
    agg = _edge_stage(xl.reshape(2 * N, HALF), meta, e8f[0])
    for l in range(1, nl):
        xl_next = _fused(agg, xl, root[l - 1].reshape(1, D), invdeg,
                         gamma[l - 1].reshape(1, D),
                         beta[l - 1].reshape(1, D), W[l], b[l].reshape(1, D))
        xl = xl_next
        agg = _edge_stage(xl.reshape(2 * N, HALF), meta, e8f[l])

    return _fused_final(agg, xl, root[nl - 1].reshape(1, D), invdeg,
                        gamma[nl - 1].reshape(1, D),
                        beta[nl - 1].reshape(1, D))


# speedup vs baseline: 1.0424x; 1.0008x over previous
import functools

import jax
import jax.numpy as jnp
import numpy as np
from jax import lax
from jax.experimental import pallas as pl
from jax.experimental.pallas import tpu as pltpu
from jax.experimental.pallas import tpu_sc as plsc

N = 10000
E = 160000
D = 256
HALF = 128
EPAD = 163840
EROWS = EPAD // 128
NP = 10240
NB_ROW = 25
RB = 400

_MESH = dict(core_axis_name="c", subcore_axis_name="s")


def _hist_body(row2d, out, idxb, valb, zb, sem, shared):
    c = lax.axis_index("c")
    s = lax.axis_index("s")
    wid = c * 16 + s

    def zfill(i, _):
        zb[pl.ds(i * 16, 16)] = jnp.zeros((16,), jnp.float32)
        return 0

    lax.fori_loop(0, 40, zfill, 0)
    pltpu.sync_copy(zb, shared.at[pl.ds(s * 640, 640)])
    pltpu.sync_copy(row2d.at[pl.ds(wid * 40, 40)], idxb)
    plsc.subcore_barrier()

    def batch(g, _):
        base = (wid * 40 + g) * 128
        for k in range(8):
            pos = base + k * 16 + lax.iota(jnp.int32, 16)
            valb[pl.ds(k * 16, 16)] = jnp.where(pos < E, 1.0, 0.0)
        pltpu.sync_copy(valb, shared.at[idxb.at[g]], add=True)
        return 0

    lax.fori_loop(0, 40, batch, 0)
    plsc.subcore_barrier()
    pltpu.sync_copy(shared.at[pl.ds(s * 640, 640)],
                    out.at[c, pl.ds(s * 640, 640)])


def _degree_hist(row2d_un):
    return pl.kernel(
        _hist_body,
        out_type=jax.ShapeDtypeStruct((2, NP), jnp.float32),
        mesh=plsc.VectorSubcoreMesh(**_MESH),
        scratch_types=[
            pltpu.VMEM((40, 128), jnp.int32),
            pltpu.VMEM((128,), jnp.float32),
            pltpu.VMEM((640,), jnp.float32),
            pltpu.SemaphoreType.DMA,
            pltpu.VMEM_SHARED((NP,), jnp.float32),
        ],
    )(row2d_un)


def _dis_body(p_ref, dis_ref, inv_ref):
    p = p_ref[...]
    deg = p[:, 0:1] + p[:, 1:2] + 1.0
    dis_ref[...] = lax.rsqrt(deg)
    inv_ref[...] = 1.0 / deg


def _dis_invdeg(partial_t):
    return pl.pallas_call(
        _dis_body,
        grid=(8,),
        in_specs=[pl.BlockSpec((NP // 8, 2), lambda j: (j, 0))],
        out_specs=[pl.BlockSpec((NP // 8, 1), lambda j: (j, 0)),
                   pl.BlockSpec((NP // 8, 1), lambda j: (j, 0))],
        out_shape=[jax.ShapeDtypeStruct((NP, 1), jnp.float32),
                   jax.ShapeDtypeStruct((NP, 1), jnp.float32)],
    )(partial_t)


def _norm_body(dis_hbm, row2d, col2d, out, disv, rb, cb, nb, sem):
    c = lax.axis_index("c")
    s = lax.axis_index("s")
    wid = c * 16 + s
    pltpu.sync_copy(dis_hbm, disv)
    pltpu.sync_copy(row2d.at[pl.ds(wid * 40, 40)], rb)
    pltpu.sync_copy(col2d.at[pl.ds(wid * 40, 40)], cb)

    def batch(g, _):
        for k in range(8):
            ri = rb[g, pl.ds(k * 16, 16)]
            ci = cb[g, pl.ds(k * 16, 16)]
            a = plsc.load_gather(disv, [ri >> 7, ri & 127])
            bb = plsc.load_gather(disv, [ci >> 7, ci & 127])
            pos = (wid * 40 + g) * 128 + k * 16 + lax.iota(jnp.int32, 16)
            nb[g, pl.ds(k * 16, 16)] = jnp.where(pos < E, a * bb, 0.0)
        return 0

    lax.fori_loop(0, 40, batch, 0)
    pltpu.sync_copy(nb, out.at[pl.ds(wid * 40, 40)])


def _edge_norm(dis_flat, row2d, col2d):
    return pl.kernel(
        _norm_body,
        out_type=jax.ShapeDtypeStruct((EROWS, 128), jnp.float32),
        compiler_params=pltpu.CompilerParams(needs_layout_passes=False),
        mesh=plsc.VectorSubcoreMesh(**_MESH),
        scratch_types=[
            pltpu.VMEM((NP // 128, 128), jnp.float32),
            pltpu.VMEM((40, 128), jnp.int32),
            pltpu.VMEM((40, 128), jnp.int32),
            pltpu.VMEM((40, 128), jnp.float32),
            pltpu.SemaphoreType.DMA,
        ],
    )(dis_flat, row2d, col2d)


def _edge_body(xlflat, meta, e8f, agg,
               rows0, rows1, mrows0, mrows1, metav, e8v,
               semg0, semg1, sems0, sems1, shared):
    c = lax.axis_index("c")
    s = lax.axis_index("s")

    pltpu.sync_copy(e8f.at[pl.ds(c * 8, 8)], e8v)

    def zfill(r, _):
        for k in range(8):
            mrows0[r, pl.ds(k * 16, 16)] = jnp.zeros((16,), jnp.float32)
        return 0

    lax.fori_loop(0, 64, zfill, 0)

    @pl.when(s < 15)
    def _():
        for i in range(10):
            pltpu.sync_copy(mrows0, shared.at[pl.ds(s * 640 + i * 64, 64)])

    @pl.when(s == 15)
    def _():
        for i in range(6):
            pltpu.sync_copy(mrows0, shared.at[pl.ds(9600 + i * 64, 64)])
        pltpu.sync_copy(mrows0.at[pl.ds(0, 16)],
                        shared.at[pl.ds(9984, 16)])

    plsc.subcore_barrier()

    lanes = [lax.iota(jnp.int32, 16) + k * 16 for k in range(8)]

    def phase_body(phase, _):
        base = s * 160 + phase * 16
        pltpu.sync_copy(meta.at[pl.ds(base * 5, 80)], metav)
        pltpu.async_copy(xlflat.at[metav.at[c]], rows0, semg0)
        pltpu.async_copy(xlflat.at[metav.at[5 + c]], rows1, semg1)

        def pair(i, _):
            for b, (rows, semg, mrows, sems) in enumerate(
                    ((rows0, semg0, mrows0, sems0),
                     (rows1, semg1, mrows1, sems1))):
                g = 2 * i + b
                pltpu.make_async_copy(xlflat.at[metav.at[5 * g + c]], rows,
                                      semg).wait()

                @pl.when(g >= 2)
                def _():
                    pltpu.make_async_copy(
                        mrows, shared.at[metav.at[2]], sems).wait()

                def grp(j, _):
                    nv16 = plsc.bitcast(metav[5 * g + 3, pl.ds(j * 16, 16)],
                                        jnp.float32)
                    ei16 = metav[5 * g + 4, pl.ds(j * 16, 16)]
                    for t in range(0, 16, 2):
                        e = j * 16 + t
                        tv0 = jnp.full((16,), t, jnp.int32)
                        tv1 = jnp.full((16,), t + 1, jnp.int32)
                        nv0 = nv16.at[tv0].get(mode="promise_in_bounds")
                        ei0 = ei16.at[tv0].get(mode="promise_in_bounds")
                        nv1 = nv16.at[tv1].get(mode="promise_in_bounds")
                        ei1 = ei16.at[tv1].get(mode="promise_in_bounds")
                        ev0 = [plsc.load_gather(e8v, [ei0, lanes[k]])
                               for k in range(8)]
                        ev1 = [plsc.load_gather(e8v, [ei1, lanes[k]])
                               for k in range(8)]
                        xv0 = [rows[e, pl.ds(k * 16, 16)] for k in range(8)]
                        xv1 = [rows[e + 1, pl.ds(k * 16, 16)]
                               for k in range(8)]
                        for k in range(8):
                            mrows[e, pl.ds(k * 16, 16)] = (
                                jnp.maximum(xv0[k] + ev0[k], 0.0) * nv0)
                        for k in range(8):
                            mrows[e + 1, pl.ds(k * 16, 16)] = (
                                jnp.maximum(xv1[k] + ev1[k], 0.0) * nv1)
                    return 0

                lax.fori_loop(0, 4, grp, 0)

                @pl.when(g < 14)
                def _():
                    pltpu.async_copy(xlflat.at[metav.at[5 * (g + 2) + c]],
                                     rows, semg)

                pltpu.async_copy(mrows, shared.at[metav.at[5 * g + 2]],
                                 sems, add=True)
            return 0

        lax.fori_loop(0, 8, pair, 0)
        pltpu.make_async_copy(mrows0, shared.at[metav.at[2]], sems0).wait()
        pltpu.make_async_copy(mrows1, shared.at[metav.at[2]], sems1).wait()
        return 0

    lax.fori_loop(0, 10, phase_body, 0)
    plsc.subcore_barrier()

    @pl.when(s < 15)
    def _():
        pltpu.sync_copy(shared.at[pl.ds(s * 640, 640)],
                        agg.at[c, pl.ds(s * 640, 640)])

    @pl.when(s == 15)
    def _():
        pltpu.sync_copy(shared.at[pl.ds(9600, 400)],
                        agg.at[c, pl.ds(9600, 400)])


def _edge_stage(xlflat, meta, e8f):
    return pl.kernel(
        _edge_body,
        out_type=jax.ShapeDtypeStruct((2, N, HALF), jnp.float32),
        compiler_params=pltpu.CompilerParams(needs_layout_passes=False),
        mesh=plsc.VectorSubcoreMesh(**_MESH),
        scratch_types=[
            pltpu.VMEM((64, 128), jnp.float32),
            pltpu.VMEM((64, 128), jnp.float32),
            pltpu.VMEM((64, 128), jnp.float32),
            pltpu.VMEM((64, 128), jnp.float32),
            pltpu.VMEM((80, 64), jnp.int32),
            pltpu.VMEM((8, 128), jnp.float32),
            pltpu.SemaphoreType.DMA,
            pltpu.SemaphoreType.DMA,
            pltpu.SemaphoreType.DMA,
            pltpu.SemaphoreType.DMA,
            pltpu.VMEM_SHARED((N, HALF), jnp.float32),
        ],
    )(xlflat, meta, e8f)


def _mm0_body(xf_ref, a_ref, c0_ref, w_ref, b_ref, out_ref):
    xf = xf_ref[...]
    h = jnp.broadcast_to(c0_ref[...], (RB, D))
    for i in range(9):
        h = h + xf[:, i:i + 1] * a_ref[i:i + 1, :]
    t = jnp.dot(h, w_ref[...], preferred_element_type=jnp.float32)
    t = t + b_ref[...]
    out_ref[0] = t[:, :HALF]
    out_ref[1] = t[:, HALF:]


def _mm0(xf, a, c0, w, bb):
    return pl.pallas_call(
        _mm0_body,
        grid=(NB_ROW,),
        in_specs=[
            pl.BlockSpec((RB, 9), lambda j: (j, 0)),
            pl.BlockSpec((9, D), lambda j: (0, 0)),
            pl.BlockSpec((1, D), lambda j: (0, 0)),
            pl.BlockSpec((D, D), lambda j: (0, 0)),
            pl.BlockSpec((1, D), lambda j: (0, 0)),
        ],
        out_specs=pl.BlockSpec((2, RB, HALF), lambda j: (0, j, 0)),
        out_shape=jax.ShapeDtypeStruct((2, N, HALF), jnp.float32),
    )(xf, a, c0, w, bb)


def _fused_body(agg_ref, xl_ref, root_ref, inv_ref, g_ref, be_ref, w_ref,
                b_ref, out_ref, outbuf, sums):
    j = pl.program_id(0)

    @pl.when(j == 0)
    def _():
        sums[...] = jnp.zeros_like(sums)

    @pl.when(j < NB_ROW)
    def _():
        a = jnp.concatenate([agg_ref[0], agg_ref[1]], axis=1)
        xl = jnp.concatenate([xl_ref[0], xl_ref[1]], axis=1)
        t = a + jnp.maximum(xl + root_ref[...], 0.0) * inv_ref[...]
        outbuf[pl.ds(j * RB, RB), :] = t
        sums[0:1, :] += jnp.sum(t, axis=0, keepdims=True)
        sums[1:2, :] += jnp.sum(t * t, axis=0, keepdims=True)

    @pl.when(j >= NB_ROW)
    def _():
        jj = j - NB_ROW
        o = outbuf[pl.ds(jj * RB, RB), :]
        mu = sums[0:1, :] / N
        var = sums[1:2, :] / N - mu * mu
        ivs = lax.rsqrt(var + 1e-5)
        h = (o - mu) * ivs * g_ref[...] + be_ref[...]
        h = jnp.maximum(h, 0.0)
        t = jnp.dot(h, w_ref[...], preferred_element_type=jnp.float32)
        t = t + b_ref[...]
        out_ref[0] = t[:, :HALF]
        out_ref[1] = t[:, HALF:]


def _fused(agg, xl, rootl, invdeg, g, be, w, bb):
    return pl.pallas_call(
        _fused_body,
        grid=(2 * NB_ROW,),
        in_specs=[
            pl.BlockSpec((2, RB, HALF),
                         lambda j: (0, jnp.where(j < NB_ROW, j, 0), 0)),
            pl.BlockSpec((2, RB, HALF),
                         lambda j: (0, jnp.where(j < NB_ROW, j, 0), 0)),
            pl.BlockSpec((1, D), lambda j: (0, 0)),
            pl.BlockSpec((RB, 1),
                         lambda j: (jnp.where(j < NB_ROW, j, 0), 0)),
            pl.BlockSpec((1, D), lambda j: (0, 0)),
            pl.BlockSpec((1, D), lambda j: (0, 0)),
            pl.BlockSpec((D, D), lambda j: (0, 0)),
            pl.BlockSpec((1, D), lambda j: (0, 0)),
        ],
        out_specs=pl.BlockSpec(
            (2, RB, HALF),
            lambda j: (0, jnp.where(j < NB_ROW, 0, j - NB_ROW), 0)),
        out_shape=jax.ShapeDtypeStruct((2, N, HALF), jnp.float32),
        scratch_shapes=[pltpu.VMEM((N, D), jnp.float32),
                        pltpu.VMEM((8, D), jnp.float32)],
    )(agg, xl, rootl, invdeg, g, be, w, bb)


def _fused_final_body(agg_ref, xl_ref, root_ref, inv_ref, g_ref, be_ref,
                      out_ref, outbuf, sums):
    j = pl.program_id(0)

    @pl.when(j == 0)
    def _():
        sums[...] = jnp.zeros_like(sums)

    @pl.when(j < NB_ROW)
    def _():
        a = jnp.concatenate([agg_ref[0], agg_ref[1]], axis=1)
        xl = jnp.concatenate([xl_ref[0], xl_ref[1]], axis=1)
        t = a + jnp.maximum(xl + root_ref[...], 0.0) * inv_ref[...]
        outbuf[pl.ds(j * RB, RB), :] = t
        sums[0:1, :] += jnp.sum(t, axis=0, keepdims=True)
        sums[1:2, :] += jnp.sum(t * t, axis=0, keepdims=True)

    @pl.when(j >= NB_ROW)
    def _():
        jj = j - NB_ROW
        o = outbuf[pl.ds(jj * RB, RB), :]
        mu = sums[0:1, :] / N
        var = sums[1:2, :] / N - mu * mu
        ivs = lax.rsqrt(var + 1e-5)
        out_ref[...] = (o - mu) * ivs * g_ref[...] + be_ref[...]


def _fused_final(agg, xl, rootl, invdeg, g, be):
    return pl.pallas_call(
        _fused_final_body,
        grid=(2 * NB_ROW,),
        in_specs=[
            pl.BlockSpec((2, RB, HALF),
                         lambda j: (0, jnp.where(j < NB_ROW, j, 0), 0)),
            pl.BlockSpec((2, RB, HALF),
                         lambda j: (0, jnp.where(j < NB_ROW, j, 0), 0)),
            pl.BlockSpec((1, D), lambda j: (0, 0)),
            pl.BlockSpec((RB, 1),
                         lambda j: (jnp.where(j < NB_ROW, j, 0), 0)),
            pl.BlockSpec((1, D), lambda j: (0, 0)),
            pl.BlockSpec((1, D), lambda j: (0, 0)),
        ],
        out_specs=pl.BlockSpec(
            (RB, D),
            lambda j: (jnp.where(j < NB_ROW, 0, j - NB_ROW), 0)),
        out_shape=jax.ShapeDtypeStruct((N, D), jnp.float32),
        scratch_shapes=[pltpu.VMEM((N, D), jnp.float32),
                        pltpu.VMEM((8, D), jnp.float32)],
    )(agg, xl, rootl, invdeg, g, be)


def kernel(x, edge_index, edge_attr, atom_tab, W, b, root, bond_tab, gamma,
           beta):
    nl = W.shape[0]
    row = edge_index[0].astype(jnp.int32)
    col = edge_index[1].astype(jnp.int32)

    pad = EPAD - E
    pad_idx = (jnp.arange(pad, dtype=jnp.int32) % N)
    rowp = jnp.concatenate([row, pad_idx])
    colp = jnp.concatenate([col, pad_idx])
    eid = (edge_attr[:, 0] + 2 * edge_attr[:, 1]
           + 4 * edge_attr[:, 2]).astype(jnp.int32)
    eidp = jnp.concatenate([eid, jnp.zeros((pad,), jnp.int32)])

    row2d = rowp.reshape(EROWS, 128)
    col2d = colp.reshape(EROWS, 128)
    row64 = rowp.reshape(EPAD // 64, 64)
    col64 = colp.reshape(EPAD // 64, 64)
    eid64 = eidp.reshape(EPAD // 64, 64)

    a_tab = (atom_tab[:, 1, :] - atom_tab[:, 0, :])
    c0 = atom_tab[:, 0, :].sum(axis=0).reshape(1, D)
    xf = x.astype(jnp.float32)

    kk = np.arange(8)
    e8 = (bond_tab[:, 0, kk & 1, :] + bond_tab[:, 1, (kk >> 1) & 1, :]
          + bond_tab[:, 2, (kk >> 2) & 1, :])
    e8f = jnp.concatenate([e8[:, :, :HALF], e8[:, :, HALF:]], axis=1)

    partial = _degree_hist(row2d)
    dis_col, invdeg = _dis_invdeg(partial.T)
    norm2d = _edge_norm(dis_col.reshape(NP // 128, 128), row2d,
                        col2d)
    normbits = lax.bitcast_convert_type(
        norm2d.reshape(EPAD // 64, 64), jnp.int32)
    meta = jnp.stack([row64, row64 + N, col64, normbits, eid64],
                     axis=1).reshape(EPAD // 64 * 5, 64)

    xl = _mm0(xf, a_tab, c0, W[0], b[0].reshape(1, D))
    agg = _edge_stage(xl.reshape(2 * N, HALF), meta, e8f[0])
    for l in range(1, nl):
        xl_next = _fused(agg, xl, root[l - 1].reshape(1, D), invdeg,
                         gamma[l - 1].reshape(1, D),
                         beta[l - 1].reshape(1, D), W[l], b[l].reshape(1, D))
        xl = xl_next
        agg = _edge_stage(xl.reshape(2 * N, HALF), meta, e8f[l])

    return _fused_final(agg, xl, root[nl - 1].reshape(1, D), invdeg,
                        gamma[nl - 1].reshape(1, D),
                        beta[nl - 1].reshape(1, D))
